# Initial kernel scaffold; baseline (speedup 1.0000x reference)
import jax
import jax.numpy as jnp
from jax.experimental import pallas as pl


def kernel(sta, src, x_grid, locs_ref, coefs, coefs_ker):
    raise NotImplementedError("write your pallas kernel here")


# speedup vs baseline: 2.3507x; 2.3507x over previous
import functools

import jax
import jax.numpy as jnp
from jax import lax
from jax.experimental import pallas as pl
from jax.experimental.pallas import tpu as pltpu
from jax.experimental.pallas import tpu_sc as plsc

K = 15
KP = 16
NSTA = 100
NREF = 500
NG = 10000
NSRC = 2000
DP = 256
SB = 256
NS_PAD = 2048
GB = 1000
SPP = 1000

NW = 32
RPW = NS_PAD // NW
CH = 4


def _softplus(x):
    return jnp.maximum(x, 0.0) + jnp.log1p(jnp.exp(-jnp.abs(x)))


def _dotT(a, b):
    return lax.dot_general(a, b, (((1,), (1,)), ((), ())),
                           preferred_element_type=jnp.float32)


def _stage_a_body(src_ref, srcrt_ref, xgT_ref, staT_ref, ckT_ref,
                  trv_ref, idx_ref, mv_ref, d2_ref):
    srcb = src_ref[...]
    srcrt = srcrt_ref[...]
    uT = 1.0 / jnp.square(_softplus(ckT_ref[...]))

    t0 = srcrt[:, 0:1] - xgT_ref[0:1, :]
    acc = t0 * t0
    t1 = srcrt[:, 1:2] - xgT_ref[1:2, :]
    acc = acc + t1 * t1
    t2d = srcrt[:, 2:3] - xgT_ref[2:3, :]
    d2_ref[...] = acc + t2d * t2d

    iota_row = lax.broadcasted_iota(jnp.int32, (SB, NG), 1)
    lane_k = lax.broadcasted_iota(jnp.int32, (SB, KP), 1)

    def body(k, carry):
        si_acc, av_acc = carry
        d2w = d2_ref[...]
        m = jnp.min(d2w, axis=1, keepdims=True)
        sel = jnp.min(jnp.where(d2w == m, iota_row, NG),
                      axis=1, keepdims=True)
        d2_ref[...] = jnp.where(iota_row == sel, jnp.float32(jnp.inf), d2w)
        onehot = (iota_row == sel).astype(jnp.float32)
        fx = _dotT(onehot, xgT_ref[0:3, :])
        fu = _dotT(onehot, uT[0:3, :])
        df = fx - srcrt
        a = -0.5 * jnp.sum(df * df * fu, axis=1, keepdims=True)
        return (jnp.where(lane_k == k, sel, si_acc),
                jnp.where(lane_k == k, a, av_acc))

    si, av = lax.fori_loop(
        0, K, body,
        (jnp.zeros((SB, KP), jnp.int32), jnp.zeros((SB, KP), jnp.float32)))
    idx_ref[...] = si

    w = jnp.where(lane_k < K, jnp.exp(av), 0.0)
    ws = jnp.sum(w, axis=1, keepdims=True)
    wn = w / jnp.where(ws == 0.0, 1.0, ws)
    sr = lax.broadcasted_iota(jnp.int32, (KP, DP), 0)
    sc16 = lax.broadcasted_iota(jnp.int32, (KP, DP), 1)
    splat = (sc16 // 16 == sr).astype(jnp.float32)
    mv_ref[...] = jnp.dot(wn, splat, preferred_element_type=jnp.float32)

    e0 = srcb[:, 0:1] - staT_ref[0:1, :]
    dd2 = e0 * e0
    e1 = srcb[:, 1:2] - staT_ref[1:2, :]
    dd2 = dd2 + e1 * e1
    e2 = srcb[:, 2:3] - staT_ref[2:3, :]
    dd2 = dd2 + e2 * e2
    dd = jnp.sqrt(dd2 + 1e-12)
    parity = lax.broadcasted_iota(jnp.int32, (SB, DP), 1) % 2
    trv_ref[...] = dd * jnp.where(parity == 0, 1.0 / 6.0, 1.0 / 3.5)


def _stage_a(src_p, src_rt_p, xgT, sta_dupT, ckT):
    nblk = NS_PAD // SB
    return pl.pallas_call(
        _stage_a_body,
        grid=(nblk,),
        in_specs=[
            pl.BlockSpec((SB, 3), lambda i: (i, 0)),
            pl.BlockSpec((SB, 3), lambda i: (i, 0)),
            pl.BlockSpec((8, NG), lambda i: (0, 0)),
            pl.BlockSpec((8, DP), lambda i: (0, 0)),
            pl.BlockSpec((8, NG), lambda i: (0, 0)),
        ],
        out_specs=[
            pl.BlockSpec((SB, DP), lambda i: (i, 0)),
            pl.BlockSpec((SB, KP), lambda i: (i, 0)),
            pl.BlockSpec((SB, DP), lambda i: (i, 0)),
        ],
        out_shape=[
            jax.ShapeDtypeStruct((NS_PAD, DP), jnp.float32),
            jax.ShapeDtypeStruct((NS_PAD, KP), jnp.int32),
            jax.ShapeDtypeStruct((NS_PAD, DP), jnp.float32),
        ],
        scratch_shapes=[
            pltpu.VMEM((SB, NG), jnp.float32),
        ],
        interpret=False,
    )(src_p, src_rt_p, xgT, sta_dupT, ckT)


def _stage_b_body(coefs_ref, sta_ref, lrT_ref,
                  xg_lo_ref, xg_hi_ref):
    sta_dup = sta_ref[...]
    q0 = sta_dup[:, 0:1] - lrT_ref[0:1, :]
    d2s = q0 * q0
    q1 = sta_dup[:, 1:2] - lrT_ref[1:2, :]
    d2s = d2s + q1 * q1
    q2 = sta_dup[:, 2:3] - lrT_ref[2:3, :]
    d2s = d2s + q2 * q2
    iota_s = lax.broadcasted_iota(jnp.int32, (DP, NREF), 1).astype(jnp.float32)
    m = jnp.min(d2s, axis=1, keepdims=True)
    sel = jnp.min(jnp.where(d2s == m, iota_s, jnp.float32(NREF)),
                  axis=1, keepdims=True)
    par_c = (lax.broadcasted_iota(jnp.int32, (DP, 1), 0) % 2).astype(jnp.float32)
    colsv = 2.0 * sel + par_c
    eye = (lax.broadcasted_iota(jnp.int32, (DP, DP), 0)
           == lax.broadcasted_iota(jnp.int32, (DP, DP), 1)).astype(jnp.float32)
    cols = lax.dot_general(colsv, eye, (((0,), (0,)), ((), ())),
                           preferred_element_type=jnp.float32)
    iota_r = lax.broadcasted_iota(jnp.int32, (SPP, DP), 0).astype(jnp.float32)
    valid = lax.broadcasted_iota(jnp.int32, (SPP, DP), 1) < 2 * NSTA
    onehot = ((iota_r == cols) & valid).astype(jnp.float32)
    xg = jnp.dot(coefs_ref[...], onehot, preferred_element_type=jnp.float32)
    xg_lo_ref[...] = xg[:, :DP // 2]
    xg_hi_ref[...] = xg[:, DP // 2:]


def _stage_b(coefs_rp, sta_dup_rt, lrT):
    nblk = NG // GB
    return pl.pallas_call(
        _stage_b_body,
        grid=(nblk,),
        in_specs=[
            pl.BlockSpec((GB, SPP), lambda i: (i, 0)),
            pl.BlockSpec((DP, 3), lambda i: (0, 0)),
            pl.BlockSpec((8, NREF), lambda i: (0, 0)),
        ],
        out_specs=[
            pl.BlockSpec((GB, DP // 2), lambda i: (i, 0)),
            pl.BlockSpec((GB, DP // 2), lambda i: (i, 0)),
        ],
        out_shape=[
            jax.ShapeDtypeStruct((NG, DP // 2), jnp.float32),
            jax.ShapeDtypeStruct((NG, DP // 2), jnp.float32),
        ],
        interpret=False,
    )(coefs_rp, sta_dup_rt, lrT)


def _stage_c_body(xg_lo_hbm, xg_hi_hbm, idxf_hbm, mv_hbm, trv_hbm, out_hbm,
                  idx_v, mv_v, trv_v, rows_lo, rows_hi, sem):
    wid = lax.axis_index("s") * 2 + lax.axis_index("c")
    base = wid * RPW
    pltpu.sync_copy(idxf_hbm.at[pl.ds(base * KP, RPW * KP)], idx_v)
    pltpu.sync_copy(mv_hbm.at[pl.ds(base, RPW)], mv_v)
    pltpu.sync_copy(trv_hbm.at[pl.ds(base, RPW)], trv_v)

    def chunk(j, carry):
        idx_sl = idx_v.at[pl.ds(j * CH * KP, CH * KP)]
        cp_lo = pltpu.async_copy(xg_lo_hbm.at[idx_sl], rows_lo, sem)
        cp_hi = pltpu.async_copy(xg_hi_hbm.at[idx_sl], rows_hi, sem)
        cp_lo.wait()
        cp_hi.wait()
        for il in range(CH):
            i = j * CH + il
            wn = [mv_v[i, pl.ds(16 * k, 16)] for k in range(K)]
            for half, rows_v, nch in ((0, rows_lo, 8), (1, rows_hi, 5)):
                for c in range(nch):
                    d = half * (DP // 2) + c * 16
                    acc = trv_v[i, pl.ds(d, 16)]
                    for k in range(K):
                        acc = acc + wn[k] * rows_v[il * KP + k,
                                                   pl.ds(c * 16, 16)]
                    trv_v[i, pl.ds(d, 16)] = acc
        return carry

    lax.fori_loop(0, RPW // CH, chunk, 0)
    pltpu.sync_copy(trv_v, out_hbm.at[pl.ds(base, RPW)])


def _stage_c(xg_lo, xg_hi, idx_flat, mv_p, trv_p):
    mesh = plsc.VectorSubcoreMesh(core_axis_name="c", subcore_axis_name="s")
    f = functools.partial(
        pl.kernel,
        out_type=jax.ShapeDtypeStruct((NS_PAD, DP), jnp.float32),
        mesh=mesh,
        compiler_params=pltpu.CompilerParams(use_tc_tiling_on_sc=True,
                                             needs_layout_passes=False),
        scratch_types=[
            pltpu.VMEM((RPW * KP,), jnp.int32),
            pltpu.VMEM((RPW, DP), jnp.float32),
            pltpu.VMEM((RPW, DP), jnp.float32),
            pltpu.VMEM((CH * KP, DP // 2), jnp.float32),
            pltpu.VMEM((CH * KP, DP // 2), jnp.float32),
            pltpu.SemaphoreType.DMA,
        ],
    )(_stage_c_body)
    return f(xg_lo, xg_hi, idx_flat, mv_p, trv_p)


def kernel(sta, src, x_grid, locs_ref, coefs, coefs_ker):
    src_rt = (src * 1000.0) / 1000.0
    xg_rt = (x_grid * 1000.0) / 1000.0
    sta_rt = (sta * 1000.0) / 1000.0
    lr_rt = (locs_ref * 1000.0) / 1000.0

    src_p = jnp.pad(src, ((0, NS_PAD - NSRC), (0, 0)))
    src_rt_p = jnp.pad(src_rt, ((0, NS_PAD - NSRC), (0, 0)))
    sta_dup = jnp.pad(jnp.repeat(sta, 2, axis=0), ((0, DP - 2 * NSTA), (0, 0)))
    sta_dup_rt = jnp.pad(jnp.repeat(sta_rt, 2, axis=0),
                         ((0, DP - 2 * NSTA), (0, 0)))
    sta_dupT = jnp.pad(sta_dup.T, ((0, 5), (0, 0)))
    xgT = jnp.pad(xg_rt.T, ((0, 5), (0, 0)))
    lrT = jnp.pad(lr_rt.T, ((0, 5), (0, 0)))
    ckT = jnp.pad(coefs_ker.T, ((0, 5), (0, 0)))
    trv_p, idx_p, mv_p = _stage_a(src_p, src_rt_p, xgT, sta_dupT, ckT)
    xg_lo, xg_hi = _stage_b(coefs.reshape(NG, 2 * NREF), sta_dup_rt, lrT)
    idx_flat = idx_p.reshape(-1)
    res = _stage_c(xg_lo, xg_hi, idx_flat, mv_p, trv_p)
    return res[:NSRC, :2 * NSTA].reshape(NSRC, NSTA, 2)
